# baseline (device time: 151906 ns/iter reference)
import functools

import jax
import jax.numpy as jnp
from jax import lax
from jax.experimental import pallas as pl
from jax.experimental.pallas import tpu as pltpu

N_DEV = 16


def _signal_all(sem, my):
    for off in range(1, N_DEV):
        peer = lax.rem(my + off, N_DEV)
        pl.semaphore_signal(
            sem, inc=1, device_id=(peer,), device_id_type=pl.DeviceIdType.MESH
        )


def _counts_allgather(cnt_row):

    def body(c_ref, out_ref, send_sem, recv_sem):
        my = lax.axis_index("i")

        barrier = pltpu.get_barrier_semaphore()
        _signal_all(barrier, my)
        pl.semaphore_wait(barrier, N_DEV - 1)

        out_ref[pl.ds(my, 1), :] = c_ref[...]

        for off in range(1, N_DEV):
            peer = lax.rem(my + off, N_DEV)
            rdma = pltpu.make_async_remote_copy(
                src_ref=c_ref,
                dst_ref=out_ref.at[pl.ds(my, 1)],
                send_sem=send_sem,
                recv_sem=recv_sem,
                device_id=(peer,),
                device_id_type=pl.DeviceIdType.MESH,
            )
            rdma.start()

        drain = pltpu.make_async_remote_copy(
            src_ref=c_ref,
            dst_ref=out_ref.at[pl.ds(my, 1)],
            send_sem=send_sem,
            recv_sem=recv_sem,
            device_id=(my,),
            device_id_type=pl.DeviceIdType.MESH,
        )
        for _ in range(N_DEV - 1):
            drain.wait_send()
        for _ in range(N_DEV - 1):
            drain.wait_recv()

        @functools.partial(pl.run_scoped, exit_sem=pltpu.SemaphoreType.REGULAR)
        def _(exit_sem):
            _signal_all(exit_sem, my)
            pl.semaphore_wait(exit_sem, N_DEV - 1)

    return pl.pallas_call(
        body,
        out_shape=jax.ShapeDtypeStruct((N_DEV, 128), jnp.int32),
        in_specs=[pl.BlockSpec(memory_space=pltpu.VMEM)],
        out_specs=pl.BlockSpec(memory_space=pltpu.VMEM),
        scratch_shapes=[pltpu.SemaphoreType.DMA, pltpu.SemaphoreType.DMA],
        compiler_params=pltpu.CompilerParams(collective_id=0),
    )(cnt_row)


def _a2av_data(x_sorted, send_cnt, send_src_off, send_dst_off):
    rows, feat = x_sorted.shape

    def body(x_ref, sc_ref, sso_ref, sdo_ref, out_ref, send_sem, recv_sem):
        my = lax.axis_index("i")

        barrier = pltpu.get_barrier_semaphore()
        _signal_all(barrier, my)
        pl.semaphore_wait(barrier, N_DEV - 1)

        for j in range(N_DEV):
            d = lax.rem(my + j, N_DEV)
            n_d = sc_ref[d]
            s0 = sso_ref[d]
            t0 = sdo_ref[d]

            def send_one(k, carry, d=d, s0=s0, t0=t0):
                rdma = pltpu.make_async_remote_copy(
                    src_ref=x_ref.at[pl.ds(s0 + k, 1)],
                    dst_ref=out_ref.at[pl.ds(t0 + k, 1)],
                    send_sem=send_sem,
                    recv_sem=recv_sem,
                    device_id=(d,),
                    device_id_type=pl.DeviceIdType.MESH,
                )
                rdma.start()
                return carry

            lax.fori_loop(0, n_d, send_one, 0)

        def wait_send_one(k, carry):
            pltpu.make_async_remote_copy(
                src_ref=x_ref.at[pl.ds(0, 1)],
                dst_ref=out_ref.at[pl.ds(0, 1)],
                send_sem=send_sem,
                recv_sem=recv_sem,
                device_id=(my,),
                device_id_type=pl.DeviceIdType.MESH,
            ).wait_send()
            return carry

        def wait_recv_one(k, carry):
            pltpu.make_async_remote_copy(
                src_ref=x_ref.at[pl.ds(0, 1)],
                dst_ref=out_ref.at[pl.ds(0, 1)],
                send_sem=send_sem,
                recv_sem=recv_sem,
                device_id=(my,),
                device_id_type=pl.DeviceIdType.MESH,
            ).wait_recv()
            return carry

        lax.fori_loop(0, rows, wait_send_one, 0)
        lax.fori_loop(0, rows, wait_recv_one, 0)

        @functools.partial(pl.run_scoped, exit_sem=pltpu.SemaphoreType.REGULAR)
        def _(exit_sem):
            _signal_all(exit_sem, my)
            pl.semaphore_wait(exit_sem, N_DEV - 1)

    return pl.pallas_call(
        body,
        out_shape=jax.ShapeDtypeStruct((rows, feat), x_sorted.dtype),
        in_specs=[
            pl.BlockSpec(memory_space=pltpu.VMEM),
            pl.BlockSpec(memory_space=pltpu.SMEM),
            pl.BlockSpec(memory_space=pltpu.SMEM),
            pl.BlockSpec(memory_space=pltpu.SMEM),
        ],
        out_specs=pl.BlockSpec(memory_space=pltpu.VMEM),
        scratch_shapes=[pltpu.SemaphoreType.DMA, pltpu.SemaphoreType.DMA],
        compiler_params=pltpu.CompilerParams(collective_id=1),
    )(x_sorted, send_cnt, send_src_off, send_dst_off)


def kernel(x, dest):
    my = lax.axis_index("i")

    order = jnp.argsort(dest)
    x_sorted = jnp.take(x, order, axis=0)

    counts_local = jnp.sum(
        dest[None, :] == jnp.arange(N_DEV, dtype=dest.dtype)[:, None],
        axis=1,
        dtype=jnp.int32,
    )
    cnt_row = jnp.zeros((1, 128), jnp.int32).at[0, :N_DEV].set(counts_local)

    cmat = _counts_allgather(cnt_row)[:, :N_DEV]

    send_cnt = counts_local
    send_src_off = jnp.cumsum(send_cnt) - send_cnt
    before_me = jnp.arange(N_DEV, dtype=jnp.int32)[:, None] < my
    send_dst_off = jnp.sum(jnp.where(before_me, cmat, 0), axis=0, dtype=jnp.int32)

    return _a2av_data(x_sorted, send_cnt, send_src_off, send_dst_off)


# device time: 130084 ns/iter; 1.1678x vs baseline; 1.1678x over previous
import functools

import jax
import jax.numpy as jnp
from jax import lax
from jax.experimental import pallas as pl
from jax.experimental.pallas import tpu as pltpu

N_DEV = 16


def _signal_all(sem, my):
    for off in range(1, N_DEV):
        peer = lax.rem(my + off, N_DEV)
        pl.semaphore_signal(
            sem, inc=1, device_id=(peer,), device_id_type=pl.DeviceIdType.MESH
        )


def _counts_allgather(cnt_row):

    def body(c_ref, out_ref, send_sem, recv_sem):
        my = lax.axis_index("i")

        barrier = pltpu.get_barrier_semaphore()
        _signal_all(barrier, my)
        pl.semaphore_wait(barrier, N_DEV - 1)

        out_ref[pl.ds(my, 1), :] = c_ref[...]

        for off in range(1, N_DEV):
            peer = lax.rem(my + off, N_DEV)
            rdma = pltpu.make_async_remote_copy(
                src_ref=c_ref,
                dst_ref=out_ref.at[pl.ds(my, 1)],
                send_sem=send_sem,
                recv_sem=recv_sem,
                device_id=(peer,),
                device_id_type=pl.DeviceIdType.MESH,
            )
            rdma.start()

        drain = pltpu.make_async_remote_copy(
            src_ref=c_ref,
            dst_ref=out_ref.at[pl.ds(my, 1)],
            send_sem=send_sem,
            recv_sem=recv_sem,
            device_id=(my,),
            device_id_type=pl.DeviceIdType.MESH,
        )
        for _ in range(N_DEV - 1):
            drain.wait_send()
        for _ in range(N_DEV - 1):
            drain.wait_recv()

        @functools.partial(pl.run_scoped, exit_sem=pltpu.SemaphoreType.REGULAR)
        def _(exit_sem):
            _signal_all(exit_sem, my)
            pl.semaphore_wait(exit_sem, N_DEV - 1)

    return pl.pallas_call(
        body,
        out_shape=jax.ShapeDtypeStruct((N_DEV, 128), jnp.int32),
        in_specs=[pl.BlockSpec(memory_space=pltpu.VMEM)],
        out_specs=pl.BlockSpec(memory_space=pltpu.VMEM),
        scratch_shapes=[pltpu.SemaphoreType.DMA, pltpu.SemaphoreType.DMA],
        compiler_params=pltpu.CompilerParams(collective_id=0),
    )(cnt_row)


def _a2av_data(x, dest, send_dst_off):
    rows, feat = x.shape

    def body(x_ref, dest_ref, sdo_ref, out_ref, ctr_ref, send_sem, recv_sem):
        my = lax.axis_index("i")

        barrier = pltpu.get_barrier_semaphore()
        _signal_all(barrier, my)
        pl.semaphore_wait(barrier, N_DEV - 1)

        for d in range(N_DEV):
            ctr_ref[d] = 0

        def send_one(j, carry):
            d = dest_ref[j]
            c = ctr_ref[d]
            ctr_ref[d] = c + 1
            rdma = pltpu.make_async_remote_copy(
                src_ref=x_ref.at[pl.ds(j, 1)],
                dst_ref=out_ref.at[pl.ds(sdo_ref[d] + c, 1)],
                send_sem=send_sem,
                recv_sem=recv_sem,
                device_id=(d,),
                device_id_type=pl.DeviceIdType.MESH,
            )
            rdma.start()
            return carry

        lax.fori_loop(0, rows, send_one, 0)

        def wait_send_one(k, carry):
            pltpu.make_async_remote_copy(
                src_ref=x_ref.at[pl.ds(0, 1)],
                dst_ref=out_ref.at[pl.ds(0, 1)],
                send_sem=send_sem,
                recv_sem=recv_sem,
                device_id=(my,),
                device_id_type=pl.DeviceIdType.MESH,
            ).wait_send()
            return carry

        def wait_recv_one(k, carry):
            pltpu.make_async_remote_copy(
                src_ref=x_ref.at[pl.ds(0, 1)],
                dst_ref=out_ref.at[pl.ds(0, 1)],
                send_sem=send_sem,
                recv_sem=recv_sem,
                device_id=(my,),
                device_id_type=pl.DeviceIdType.MESH,
            ).wait_recv()
            return carry

        lax.fori_loop(0, rows, wait_send_one, 0)
        lax.fori_loop(0, rows, wait_recv_one, 0)

        @functools.partial(pl.run_scoped, exit_sem=pltpu.SemaphoreType.REGULAR)
        def _(exit_sem):
            _signal_all(exit_sem, my)
            pl.semaphore_wait(exit_sem, N_DEV - 1)

    return pl.pallas_call(
        body,
        out_shape=jax.ShapeDtypeStruct((rows, feat), x.dtype),
        in_specs=[
            pl.BlockSpec(memory_space=pltpu.VMEM),
            pl.BlockSpec(memory_space=pltpu.SMEM),
            pl.BlockSpec(memory_space=pltpu.SMEM),
        ],
        out_specs=pl.BlockSpec(memory_space=pltpu.VMEM),
        scratch_shapes=[
            pltpu.SMEM((N_DEV,), jnp.int32),
            pltpu.SemaphoreType.DMA,
            pltpu.SemaphoreType.DMA,
        ],
        compiler_params=pltpu.CompilerParams(collective_id=1),
    )(x, dest, send_dst_off)


def kernel(x, dest):
    my = lax.axis_index("i")

    counts_local = jnp.sum(
        dest[None, :] == jnp.arange(N_DEV, dtype=dest.dtype)[:, None],
        axis=1,
        dtype=jnp.int32,
    )
    cnt_row = jnp.zeros((1, 128), jnp.int32).at[0, :N_DEV].set(counts_local)

    cmat = _counts_allgather(cnt_row)[:, :N_DEV]

    before_me = jnp.arange(N_DEV, dtype=jnp.int32)[:, None] < my
    send_dst_off = jnp.sum(jnp.where(before_me, cmat, 0), axis=0, dtype=jnp.int32)

    return _a2av_data(x, dest.astype(jnp.int32), send_dst_off)
